# Initial kernel scaffold; baseline (speedup 1.0000x reference)
import jax
import jax.numpy as jnp
from jax.experimental import pallas as pl


def kernel(x, codebook):
    raise NotImplementedError("write your pallas kernel here")


# speedup vs baseline: 1.0172x; 1.0172x over previous
import jax
import jax.numpy as jnp
from jax.experimental import pallas as pl

BATCH = 256
DIM_CODES = 32
BOOK_SIZE = 1024
EMBEDDING_DIM = 256


def _vq_kernel(x_ref, cb_ref, dist_ref, idx_ref):
    d = pl.program_id(0)
    x_d = x_ref[:, d, :]
    cb_d = cb_ref[0]
    cross = jax.lax.dot_general(
        x_d, cb_d,
        dimension_numbers=(((1,), (1,)), ((), ())),
        preferred_element_type=jnp.float32,
    )
    x_sq = jnp.sum(x_d * x_d, axis=-1, keepdims=True)
    c_sq = jnp.sum(cb_d * cb_d, axis=-1)[None, :]
    dist = x_sq + c_sq - 2.0 * cross
    dist_ref[...] = dist
    m = jnp.min(dist, axis=-1, keepdims=True)
    iota = jax.lax.broadcasted_iota(jnp.int32, dist.shape, 1)
    idx = jnp.min(jnp.where(dist == m, iota, BOOK_SIZE), axis=-1)
    idx_ref[0, 0, :] = idx


def kernel(x, codebook):
    batch = x.shape[0]
    dim_codes, book_size, emb = codebook.shape
    x3 = x.reshape(batch, dim_codes, emb)
    dist, idx_t = pl.pallas_call(
        _vq_kernel,
        grid=(dim_codes,),
        in_specs=[
            pl.BlockSpec((batch, dim_codes, emb), lambda d: (0, 0, 0)),
            pl.BlockSpec((1, book_size, emb), lambda d: (d, 0, 0)),
        ],
        out_specs=[
            pl.BlockSpec((batch, book_size), lambda d: (0, d)),
            pl.BlockSpec((1, 1, batch), lambda d: (d, 0, 0)),
        ],
        out_shape=[
            jax.ShapeDtypeStruct((batch, dim_codes * book_size), jnp.float32),
            jax.ShapeDtypeStruct((dim_codes, 1, batch), jnp.int32),
        ],
    )(x3, codebook)
    dist = dist.reshape(batch, dim_codes, book_size)
    idx = idx_t.reshape(dim_codes, batch).T.reshape(batch, dim_codes, 1)
    return dist, idx.astype(jnp.int64)


# speedup vs baseline: 1.8493x; 1.8181x over previous
import jax
import jax.numpy as jnp
from jax.experimental import pallas as pl

BATCH = 256
DIM_CODES = 32
BOOK_SIZE = 1024
EMBEDDING_DIM = 256
D_BLK = 8


def _vq_kernel(x_ref, cb_ref, dist_ref, idx_ref):
    for j in range(D_BLK):
        x_d = x_ref[:, j * EMBEDDING_DIM:(j + 1) * EMBEDDING_DIM]
        cb_d = cb_ref[j]
        cross = jax.lax.dot_general(
            x_d, cb_d,
            dimension_numbers=(((1,), (1,)), ((), ())),
            preferred_element_type=jnp.float32,
        )
        x_sq = jnp.sum(x_d * x_d, axis=-1, keepdims=True)
        c_sq = jnp.sum(cb_d * cb_d, axis=-1)[None, :]
        dist = x_sq + c_sq - 2.0 * cross
        dist_ref[:, j, :] = dist
        m = jnp.min(dist, axis=-1, keepdims=True)
        iota = jax.lax.broadcasted_iota(jnp.int32, dist.shape, 1)
        idx = jnp.min(jnp.where(dist == m, iota, BOOK_SIZE), axis=-1)
        idx_ref[:, j, 0] = idx


def kernel(x, codebook):
    batch = x.shape[0]
    dim_codes, book_size, emb = codebook.shape
    n_grid = dim_codes // D_BLK
    dist, idx = pl.pallas_call(
        _vq_kernel,
        grid=(n_grid,),
        in_specs=[
            pl.BlockSpec((batch, D_BLK * emb), lambda g: (0, g)),
            pl.BlockSpec((D_BLK, book_size, emb), lambda g: (g, 0, 0)),
        ],
        out_specs=[
            pl.BlockSpec((batch, D_BLK, book_size), lambda g: (0, g, 0)),
            pl.BlockSpec((batch, D_BLK, 1), lambda g: (0, g, 0)),
        ],
        out_shape=[
            jax.ShapeDtypeStruct((batch, dim_codes, book_size), jnp.float32),
            jax.ShapeDtypeStruct((batch, dim_codes, 1), jnp.int32),
        ],
    )(x, codebook)
    return dist, idx.astype(jnp.int64)
